# baseline (speedup 1.0000x reference)
import functools

import jax
import jax.numpy as jnp
from jax import lax
from jax.experimental import pallas as pl
from jax.experimental.pallas import tpu as pltpu
from jax.experimental.pallas import tpu_sc as plsc

BATCH = 16384
EMBED_DIM = 32
NUM_CORES = 2
NUM_SUBCORES = 16
LANES = 16
NUM_WORKERS = NUM_CORES * NUM_SUBCORES
BW = BATCH // NUM_WORKERS
IDX_CHUNK = 128
N_CHUNKS = BW // IDX_CHUNK
GROUPS = BW // LANES

_mesh = plsc.VectorSubcoreMesh(core_axis_name="c", subcore_axis_name="s")


@functools.partial(
    pl.kernel,
    mesh=_mesh,
    out_type=jax.ShapeDtypeStruct((BATCH,), jnp.float32),
    compiler_params=pltpu.CompilerParams(
        needs_layout_passes=False, use_tc_tiling_on_sc=False),
    scratch_types=[
        pltpu.VMEM((BW,), jnp.int32),
        pltpu.VMEM((BW,), jnp.int32),
        pltpu.VMEM((BW, EMBED_DIM), jnp.bfloat16),
        pltpu.VMEM((BW, EMBED_DIM), jnp.bfloat16),
        pltpu.VMEM((BW, EMBED_DIM), jnp.float32),
        pltpu.VMEM((BW, EMBED_DIM), jnp.float32),
        pltpu.VMEM((BW,), jnp.float32),
        pltpu.VMEM((BW,), jnp.float32),
        pltpu.VMEM((BW,), jnp.float32),
        pltpu.SemaphoreType.DMA,
    ],
)
def _mf_sc_kernel(uid_hbm, iid_hbm, ue_hbm, ub_hbm, ie_hbm, ib_hbm, out_hbm,
                  uid_v, iid_v, p_bf, q_bf, p_v, q_v, pb_v, qb_v, out_v, sem):
    wid = lax.axis_index("s") * NUM_CORES + lax.axis_index("c")
    base = wid * BW

    pltpu.sync_copy(uid_hbm.at[pl.ds(base, BW)], uid_v)
    pltpu.sync_copy(iid_hbm.at[pl.ds(base, BW)], iid_v)

    copies = []
    for c in range(N_CHUNKS):
        s = pl.ds(c * IDX_CHUNK, IDX_CHUNK)
        copies.append(pltpu.async_copy(ue_hbm.at[uid_v.at[s]], p_bf.at[s], sem))
        copies.append(pltpu.async_copy(ie_hbm.at[iid_v.at[s]], q_bf.at[s], sem))
        copies.append(pltpu.async_copy(ub_hbm.at[uid_v.at[s]], pb_v.at[s], sem))
        copies.append(pltpu.async_copy(ib_hbm.at[iid_v.at[s]], qb_v.at[s], sem))
    for cp in copies:
        cp.wait()

    def widen_body(b, carry):
        ev, od = plsc.unpack(p_bf[b, :], format=plsc.PackFormat.INTERLEAVED)
        p_v[b, pl.ds(0, LANES)] = ev
        p_v[b, pl.ds(LANES, LANES)] = od
        ev, od = plsc.unpack(q_bf[b, :], format=plsc.PackFormat.INTERLEAVED)
        q_v[b, pl.ds(0, LANES)] = ev
        q_v[b, pl.ds(LANES, LANES)] = od
        return carry

    lax.fori_loop(0, BW, widen_body, 0)

    lanes = lax.iota(jnp.int32, LANES)

    def group_body(g, carry):
        rows = g * LANES + lanes
        acc = plsc.load_gather(pb_v, [rows])
        acc = acc + plsc.load_gather(qb_v, [rows])
        for d in range(EMBED_DIM):
            dcol = jnp.full((LANES,), d, jnp.int32)
            acc = acc + (plsc.load_gather(p_v, [rows, dcol])
                         * plsc.load_gather(q_v, [rows, dcol]))
        plsc.store_scatter(out_v, [rows], acc)
        return carry

    lax.fori_loop(0, GROUPS, group_body, 0)

    pltpu.sync_copy(out_v, out_hbm.at[pl.ds(base, BW)])


def kernel(user_id, item_id, user_embedding, user_bias, item_embedding, item_bias):
    uid = user_id.astype(jnp.int32)
    iid = item_id.astype(jnp.int32)
    return _mf_sc_kernel(uid, iid, user_embedding.astype(jnp.bfloat16),
                         user_bias.reshape(-1),
                         item_embedding.astype(jnp.bfloat16),
                         item_bias.reshape(-1))


# speedup vs baseline: 1.2201x; 1.2201x over previous
import functools

import jax
import jax.numpy as jnp
from jax import lax
from jax.experimental import pallas as pl
from jax.experimental.pallas import tpu as pltpu
from jax.experimental.pallas import tpu_sc as plsc

BATCH = 16384
EMBED_DIM = 32
NUM_CORES = 2
NUM_SUBCORES = 16
LANES = 16
NUM_WORKERS = NUM_CORES * NUM_SUBCORES
BW = BATCH // NUM_WORKERS
IDX_CHUNK = 128
N_CHUNKS = BW // IDX_CHUNK
GROUPS = BW // LANES

_mesh = plsc.VectorSubcoreMesh(core_axis_name="c", subcore_axis_name="s")


@functools.partial(
    pl.kernel,
    mesh=_mesh,
    out_type=jax.ShapeDtypeStruct((BATCH,), jnp.float32),
    compiler_params=pltpu.CompilerParams(
        needs_layout_passes=False, use_tc_tiling_on_sc=False),
    scratch_types=[
        pltpu.VMEM((BW,), jnp.int32),
        pltpu.VMEM((BW,), jnp.int32),
        pltpu.VMEM((BW, EMBED_DIM), jnp.float32),
        pltpu.VMEM((BW, EMBED_DIM), jnp.float32),
        pltpu.VMEM((BW,), jnp.float32),
        pltpu.VMEM((BW,), jnp.float32),
        pltpu.VMEM((BW,), jnp.float32),
        pltpu.SemaphoreType.DMA,
    ],
)
def _mf_sc_kernel(uid_hbm, iid_hbm, ue_hbm, ub_hbm, ie_hbm, ib_hbm, out_hbm,
                  uid_v, iid_v, p_v, q_v, pb_v, qb_v, out_v, sem):
    wid = lax.axis_index("s") * NUM_CORES + lax.axis_index("c")
    base = wid * BW

    pltpu.sync_copy(uid_hbm.at[pl.ds(base, BW)], uid_v)
    pltpu.sync_copy(iid_hbm.at[pl.ds(base, BW)], iid_v)

    copies = []
    for c in range(N_CHUNKS):
        s = pl.ds(c * IDX_CHUNK, IDX_CHUNK)
        copies.append(pltpu.async_copy(ue_hbm.at[uid_v.at[s]], p_v.at[s], sem))
        copies.append(pltpu.async_copy(ie_hbm.at[iid_v.at[s]], q_v.at[s], sem))
        copies.append(pltpu.async_copy(ub_hbm.at[uid_v.at[s]], pb_v.at[s], sem))
        copies.append(pltpu.async_copy(ib_hbm.at[iid_v.at[s]], qb_v.at[s], sem))
    for cp in copies:
        cp.wait()

    lanes = lax.iota(jnp.int32, LANES)

    def group_body(g, carry):
        rows = g * LANES + lanes
        acc = plsc.load_gather(pb_v, [rows])
        acc = acc + plsc.load_gather(qb_v, [rows])
        for d in range(EMBED_DIM):
            dcol = jnp.full((LANES,), d, jnp.int32)
            acc = acc + (plsc.load_gather(p_v, [rows, dcol])
                         * plsc.load_gather(q_v, [rows, dcol]))
        plsc.store_scatter(out_v, [rows], acc)
        return carry

    lax.fori_loop(0, GROUPS, group_body, 0)

    pltpu.sync_copy(out_v, out_hbm.at[pl.ds(base, BW)])


def kernel(user_id, item_id, user_embedding, user_bias, item_embedding, item_bias):
    uid = user_id.astype(jnp.int32)
    iid = item_id.astype(jnp.int32)
    return _mf_sc_kernel(uid, iid, user_embedding, user_bias.reshape(-1),
                         item_embedding, item_bias.reshape(-1))
